# baseline (speedup 1.0000x reference)
import functools
import math

import jax
import jax.numpy as jnp
from jax import lax
from jax.experimental import pallas as pl
from jax.experimental.pallas import tpu as pltpu
from jax.experimental.pallas import tpu_sc as plsc

D_MODEL = 64
SCALE = math.sqrt(D_MODEL)

_info = plsc.get_sparse_core_info()
NC, NS, L = _info.num_cores, _info.num_subcores, _info.num_lanes
NW = NC * NS

CHUNK = 128


def _emb_body(n_chunks, lut_hbm, idx_hbm, out_hbm, idx_v, rows_v, sem_g, sem_i):
    wid = lax.axis_index("s") * NC + lax.axis_index("c")
    row0 = wid * n_chunks

    pltpu.async_copy(idx_hbm.at[pl.ds(row0, n_chunks)], idx_v, sem_i).wait()

    def chunk_body(j, _):
        pltpu.async_copy(lut_hbm.at[idx_v.at[j]], rows_v, sem_g).wait()

        def row_body(r, _):
            for t in range(D_MODEL // L):
                sl = pl.ds(t * L, L)
                rows_v[r, sl] = rows_v[r, sl] * SCALE
            return 0

        lax.fori_loop(0, CHUNK, row_body, 0)
        out_base = (row0 + j) * CHUNK
        pltpu.async_copy(rows_v, out_hbm.at[pl.ds(out_base, CHUNK)], sem_g).wait()
        return 0

    lax.fori_loop(0, n_chunks, chunk_body, 0)


def kernel(x, lut):
    B, S = x.shape
    total = B * S
    assert total % (NW * CHUNK) == 0
    n_chunks = total // (NW * CHUNK)

    idx = x.reshape(total // CHUNK, CHUNK).astype(jnp.int32)

    mesh = plsc.VectorSubcoreMesh(core_axis_name="c", subcore_axis_name="s")
    k = functools.partial(
        pl.kernel,
        mesh=mesh,
        out_type=jax.ShapeDtypeStruct((total, D_MODEL), jnp.float32),
        scratch_types=[
            pltpu.VMEM((n_chunks, CHUNK), jnp.int32),
            pltpu.VMEM((CHUNK, D_MODEL), jnp.float32),
            pltpu.SemaphoreType.DMA,
            pltpu.SemaphoreType.DMA,
        ],
        compiler_params=pltpu.CompilerParams(use_tc_tiling_on_sc=False),
    )(functools.partial(_emb_body, n_chunks))

    out = k(lut, idx)
    return out.reshape(B, S, D_MODEL)


# speedup vs baseline: 1.2093x; 1.2093x over previous
import functools
import math

import jax
import jax.numpy as jnp
from jax import lax
from jax.experimental import pallas as pl
from jax.experimental.pallas import tpu as pltpu
from jax.experimental.pallas import tpu_sc as plsc

D_MODEL = 64
SCALE = math.sqrt(D_MODEL)

_info = plsc.get_sparse_core_info()
NC, NS, L = _info.num_cores, _info.num_subcores, _info.num_lanes
NW = NC * NS

CHUNK = 128
NBUF = 4


def _emb_body(n_chunks, lut_hbm, idx_hbm, out_hbm, idx_v, *bufs_and_sems):
    gbuf = bufs_and_sems[:NBUF]
    obuf = bufs_and_sems[NBUF:2 * NBUF]
    isem = bufs_and_sems[2 * NBUF]
    gsem = bufs_and_sems[2 * NBUF + 1:2 * NBUF + 1 + NBUF]
    ssem = bufs_and_sems[2 * NBUF + 1 + NBUF:]

    wid = lax.axis_index("s") * NC + lax.axis_index("c")
    row0 = wid * n_chunks

    pltpu.async_copy(idx_hbm.at[pl.ds(row0, n_chunks)], idx_v, isem).wait()

    def start_gather(b, j):
        pltpu.async_copy(lut_hbm.at[idx_v.at[j]], gbuf[b], gsem[b])

    def wait_gather(b, j):
        pltpu.make_async_copy(lut_hbm.at[idx_v.at[j]], gbuf[b], gsem[b]).wait()

    def start_scatter(b, j):
        dst = out_hbm.at[pl.ds((row0 + j) * CHUNK, CHUNK)]
        pltpu.async_copy(obuf[b], dst, ssem[b])

    def wait_scatter(b, j):
        dst = out_hbm.at[pl.ds((row0 + j) * CHUNK, CHUNK)]
        pltpu.make_async_copy(obuf[b], dst, ssem[b]).wait()

    def scale(b):
        g, o = gbuf[b], obuf[b]

        @plsc.parallel_loop(0, CHUNK, 1, unroll=8)
        def _(r):
            for t in range(D_MODEL // L):
                sl = pl.ds(t * L, L)
                o[r, sl] = g[r, sl] * SCALE

    for b in range(NBUF):
        start_gather(b, b)

    for b in range(NBUF):
        wait_gather(b, b)
        scale(b)
        start_gather(b, b + NBUF)
        start_scatter(b, b)

    n_groups = n_chunks // NBUF

    def group_body(g, _):
        for b in range(NBUF):
            j = g * NBUF + b
            wait_gather(b, j)
            wait_scatter(b, j - NBUF)
            scale(b)
            start_gather(b, j + NBUF)
            start_scatter(b, j)
        return 0

    lax.fori_loop(1, n_groups - 1, group_body, 0)

    for b in range(NBUF):
        j = (n_groups - 1) * NBUF + b
        wait_gather(b, j)
        wait_scatter(b, j - NBUF)
        scale(b)
        start_scatter(b, j)

    for b in range(NBUF):
        j = (n_groups - 1) * NBUF + b
        wait_scatter(b, j)


def kernel(x, lut):
    B, S = x.shape
    total = B * S
    assert total % (NW * CHUNK * NBUF) == 0
    n_chunks = total // (NW * CHUNK)

    idx = x.reshape(total // CHUNK, CHUNK).astype(jnp.int32)

    scratch = [pltpu.VMEM((n_chunks, CHUNK), jnp.int32)]
    scratch += [pltpu.VMEM((CHUNK, D_MODEL), jnp.float32) for _ in range(2 * NBUF)]
    scratch += [pltpu.SemaphoreType.DMA for _ in range(1 + 2 * NBUF)]

    mesh = plsc.VectorSubcoreMesh(core_axis_name="c", subcore_axis_name="s")
    k = functools.partial(
        pl.kernel,
        mesh=mesh,
        out_type=jax.ShapeDtypeStruct((total, D_MODEL), jnp.float32),
        scratch_types=scratch,
        compiler_params=pltpu.CompilerParams(use_tc_tiling_on_sc=False),
    )(functools.partial(_emb_body, n_chunks))

    out = k(lut, idx)
    return out.reshape(B, S, D_MODEL)


# speedup vs baseline: 1.4465x; 1.1961x over previous
import functools
import math

import jax
import jax.numpy as jnp
from jax import lax
from jax.experimental import pallas as pl
from jax.experimental.pallas import tpu as pltpu
from jax.experimental.pallas import tpu_sc as plsc

D_MODEL = 64
SCALE = math.sqrt(D_MODEL)

_info = plsc.get_sparse_core_info()
NC, NS, L = _info.num_cores, _info.num_subcores, _info.num_lanes
NW = NC * NS

CHUNK = 40
NBUF = 4


def _emb_body(rows_per_w, seq, lut_hbm, x_hbm, out_hbm, idx_v, *bufs_and_sems):
    gbuf = bufs_and_sems[:NBUF]
    obuf = bufs_and_sems[NBUF:2 * NBUF]
    isem = bufs_and_sems[2 * NBUF]
    gsem = bufs_and_sems[2 * NBUF + 1:2 * NBUF + 1 + NBUF]
    ssem = bufs_and_sems[2 * NBUF + 1 + NBUF:]

    halves = seq // CHUNK
    n_chunks = rows_per_w * halves

    wid = lax.axis_index("s") * NC + lax.axis_index("c")
    row0 = wid * rows_per_w

    pltpu.async_copy(x_hbm.at[pl.ds(row0, rows_per_w)], idx_v, isem).wait()

    def idx_slice(j):
        r = j // halves
        off = (j % halves) * CHUNK
        return idx_v.at[r, pl.ds(off, CHUNK)]

    def out_slice(j):
        r = j // halves
        off = (j % halves) * CHUNK
        return out_hbm.at[row0 + r, pl.ds(off, CHUNK)]

    def start_gather(b, j):
        pltpu.async_copy(lut_hbm.at[idx_slice(j)], gbuf[b], gsem[b])

    def wait_gather(b, j):
        pltpu.make_async_copy(lut_hbm.at[idx_slice(j)], gbuf[b], gsem[b]).wait()

    def start_scatter(b, j):
        pltpu.async_copy(obuf[b], out_slice(j), ssem[b])

    def wait_scatter(b, j):
        pltpu.make_async_copy(obuf[b], out_slice(j), ssem[b]).wait()

    def scale(b):
        g, o = gbuf[b], obuf[b]

        @plsc.parallel_loop(0, CHUNK, 1, unroll=8)
        def _(r):
            for t in range(D_MODEL // L):
                sl = pl.ds(t * L, L)
                o[r, sl] = g[r, sl] * SCALE

    for b in range(NBUF):
        start_gather(b, b)

    for b in range(NBUF):
        wait_gather(b, b)
        scale(b)
        start_gather(b, b + NBUF)
        start_scatter(b, b)

    n_groups = n_chunks // NBUF

    def group_body(g, _):
        for b in range(NBUF):
            j = g * NBUF + b
            wait_gather(b, j)
            wait_scatter(b, j - NBUF)
            scale(b)
            start_gather(b, j + NBUF)
            start_scatter(b, j)
        return 0

    lax.fori_loop(1, n_groups - 1, group_body, 0)

    for b in range(NBUF):
        j = (n_groups - 1) * NBUF + b
        wait_gather(b, j)
        wait_scatter(b, j - NBUF)
        scale(b)
        start_scatter(b, j)

    for b in range(NBUF):
        j = (n_groups - 1) * NBUF + b
        wait_scatter(b, j)


def kernel(x, lut):
    B, S = x.shape
    assert B % NW == 0 and S % CHUNK == 0
    rows_per_w = B // NW
    assert (rows_per_w * (S // CHUNK)) % NBUF == 0

    scratch = [pltpu.VMEM((rows_per_w, S), jnp.int32)]
    scratch += [pltpu.VMEM((CHUNK, D_MODEL), jnp.float32) for _ in range(2 * NBUF)]
    scratch += [pltpu.SemaphoreType.DMA for _ in range(1 + 2 * NBUF)]

    mesh = plsc.VectorSubcoreMesh(core_axis_name="c", subcore_axis_name="s")
    k = functools.partial(
        pl.kernel,
        mesh=mesh,
        out_type=jax.ShapeDtypeStruct((B, S, D_MODEL), jnp.float32),
        scratch_types=scratch,
        compiler_params=pltpu.CompilerParams(use_tc_tiling_on_sc=False),
    )(functools.partial(_emb_body, rows_per_w, S))

    return k(lut, x)


# speedup vs baseline: 1.4717x; 1.0175x over previous
import functools
import math

import jax
import jax.numpy as jnp
from jax import lax
from jax.experimental import pallas as pl
from jax.experimental.pallas import tpu as pltpu
from jax.experimental.pallas import tpu_sc as plsc

D_MODEL = 64
D_PAD = 128
SCALE = math.sqrt(D_MODEL)

_info = plsc.get_sparse_core_info()
NC, NS, L = _info.num_cores, _info.num_subcores, _info.num_lanes
NW = NC * NS

GCHUNK = 40
NBUF = 4


def _emb_body(rows_per_w, seq, lut_hbm, x_hbm, out_hbm, idx_v, *bufs_and_sems):
    rbuf = bufs_and_sems[:NBUF]
    isem = bufs_and_sems[NBUF]
    gsem = bufs_and_sems[NBUF + 1:NBUF + 1 + NBUF]
    ssem = bufs_and_sems[NBUF + 1 + NBUF:]

    halves = seq // GCHUNK

    wid = lax.axis_index("s") * NC + lax.axis_index("c")
    row0 = wid * rows_per_w

    pltpu.async_copy(x_hbm.at[pl.ds(row0, rows_per_w)], idx_v, isem).wait()

    def start_gathers(b, r):
        for h in range(halves):
            pltpu.async_copy(
                lut_hbm.at[idx_v.at[r, pl.ds(h * GCHUNK, GCHUNK)]],
                rbuf[b].at[pl.ds(h * GCHUNK, GCHUNK)],
                gsem[b],
            )

    def wait_gathers(b, r):
        for h in range(halves):
            pltpu.make_async_copy(
                lut_hbm.at[idx_v.at[r, pl.ds(h * GCHUNK, GCHUNK)]],
                rbuf[b].at[pl.ds(h * GCHUNK, GCHUNK)],
                gsem[b],
            ).wait()

    def start_scatter(b, r):
        pltpu.async_copy(rbuf[b], out_hbm.at[row0 + r], ssem[b])

    def wait_scatter(b, r):
        pltpu.make_async_copy(rbuf[b], out_hbm.at[row0 + r], ssem[b]).wait()

    def scale(b):
        buf = rbuf[b]

        @plsc.parallel_loop(0, seq, 1, unroll=8)
        def _(r):
            for t in range(D_MODEL // L):
                sl = pl.ds(t * L, L)
                buf[r, sl] = buf[r, sl] * SCALE

    n_groups = rows_per_w // NBUF

    for b in range(NBUF - 1):
        start_gathers(b, b)

    for b in range(NBUF):
        wait_gathers(b, b)
        scale(b)
        start_scatter(b, b)
        bn = (b + NBUF - 1) % NBUF
        if b == 0:
            start_gathers(NBUF - 1, NBUF - 1)
        else:
            wait_scatter(bn, b - 1)
            start_gathers(bn, b + NBUF - 1)

    def group_body(g, _):
        for b in range(NBUF):
            r = g * NBUF + b
            wait_gathers(b, r)
            scale(b)
            start_scatter(b, r)
            bn = (b + NBUF - 1) % NBUF
            wait_scatter(bn, r - 1)
            start_gathers(bn, r + NBUF - 1)
        return 0

    lax.fori_loop(1, n_groups - 1, group_body, 0)

    for b in range(NBUF):
        r = (n_groups - 1) * NBUF + b
        wait_gathers(b, r)
        scale(b)
        start_scatter(b, r)
        if r + NBUF - 1 < rows_per_w:
            bn = (b + NBUF - 1) % NBUF
            wait_scatter(bn, r - 1)
            start_gathers(bn, r + NBUF - 1)

    for r in range(rows_per_w - NBUF, rows_per_w):
        wait_scatter(r % NBUF, r)


def kernel(x, lut):
    B, S = x.shape
    V, D = lut.shape
    assert B % NW == 0 and S % GCHUNK == 0 and D == D_MODEL
    rows_per_w = B // NW

    lut_pad = jnp.pad(lut, ((0, 0), (0, D_PAD - D)))

    scratch = [pltpu.VMEM((rows_per_w, S), jnp.int32)]
    scratch += [pltpu.VMEM((S, D_PAD), jnp.float32) for _ in range(NBUF)]
    scratch += [pltpu.SemaphoreType.DMA for _ in range(1 + 2 * NBUF)]

    mesh = plsc.VectorSubcoreMesh(core_axis_name="c", subcore_axis_name="s")
    k = functools.partial(
        pl.kernel,
        mesh=mesh,
        out_type=jax.ShapeDtypeStruct((B, S, D_PAD), jnp.float32),
        scratch_types=scratch,
        compiler_params=pltpu.CompilerParams(use_tc_tiling_on_sc=False),
    )(functools.partial(_emb_body, rows_per_w, S))

    out = k(lut_pad, x)
    return out[..., :D_MODEL]


# speedup vs baseline: 1.7250x; 1.1721x over previous
import functools
import math

import jax
import jax.numpy as jnp
from jax import lax
from jax.experimental import pallas as pl
from jax.experimental.pallas import tpu as pltpu
from jax.experimental.pallas import tpu_sc as plsc

D_MODEL = 64
D_PAD = 128
SCALE = math.sqrt(D_MODEL)

_info = plsc.get_sparse_core_info()
NC, NS, L = _info.num_cores, _info.num_subcores, _info.num_lanes
NW = NC * NS

GCHUNK = 40
NBUF = 4


def _emb_body(rows_per_w, seq, lut_hbm, x_hbm, out_hbm, idx_v, *bufs_and_sems):
    rbuf = bufs_and_sems[:NBUF]
    isem = bufs_and_sems[NBUF]
    gsem = bufs_and_sems[NBUF + 1:NBUF + 1 + NBUF]
    ssem = bufs_and_sems[NBUF + 1 + NBUF:]

    halves = seq // GCHUNK

    wid = lax.axis_index("s") * NC + lax.axis_index("c")
    row0 = wid * rows_per_w

    pltpu.async_copy(x_hbm.at[pl.ds(row0, rows_per_w)], idx_v, isem).wait()

    def start_gathers(b, r):
        for h in range(halves):
            pltpu.async_copy(
                lut_hbm.at[idx_v.at[r, pl.ds(h * GCHUNK, GCHUNK)]],
                rbuf[b].at[pl.ds(h * GCHUNK, GCHUNK)],
                gsem[b],
            )

    def wait_gathers(b, r):
        for h in range(halves):
            pltpu.make_async_copy(
                lut_hbm.at[idx_v.at[r, pl.ds(h * GCHUNK, GCHUNK)]],
                rbuf[b].at[pl.ds(h * GCHUNK, GCHUNK)],
                gsem[b],
            ).wait()

    def start_scatter(b, r):
        pltpu.async_copy(
            rbuf[b], out_hbm.at[row0 + r, :, pl.ds(0, D_MODEL)], ssem[b])

    def wait_scatter(b, r):
        pltpu.make_async_copy(
            rbuf[b], out_hbm.at[row0 + r, :, pl.ds(0, D_MODEL)], ssem[b]).wait()

    def scale(b):
        buf = rbuf[b]

        @plsc.parallel_loop(0, seq, 1, unroll=8)
        def _(r):
            for t in range(D_MODEL // L):
                sl = pl.ds(t * L, L)
                buf[r, sl] = buf[r, sl] * SCALE

    n_groups = rows_per_w // NBUF

    for b in range(NBUF - 1):
        start_gathers(b, b)

    for b in range(NBUF):
        wait_gathers(b, b)
        scale(b)
        start_scatter(b, b)
        bn = (b + NBUF - 1) % NBUF
        if b == 0:
            start_gathers(NBUF - 1, NBUF - 1)
        else:
            wait_scatter(bn, b - 1)
            start_gathers(bn, b + NBUF - 1)

    def group_body(g, _):
        for b in range(NBUF):
            r = g * NBUF + b
            wait_gathers(b, r)
            scale(b)
            start_scatter(b, r)
            bn = (b + NBUF - 1) % NBUF
            wait_scatter(bn, r - 1)
            start_gathers(bn, r + NBUF - 1)
        return 0

    lax.fori_loop(1, n_groups - 1, group_body, 0)

    for b in range(NBUF):
        r = (n_groups - 1) * NBUF + b
        wait_gathers(b, r)
        scale(b)
        start_scatter(b, r)
        if r + NBUF - 1 < rows_per_w:
            bn = (b + NBUF - 1) % NBUF
            wait_scatter(bn, r - 1)
            start_gathers(bn, r + NBUF - 1)

    for r in range(rows_per_w - NBUF, rows_per_w):
        wait_scatter(r % NBUF, r)


def kernel(x, lut):
    B, S = x.shape
    V, D = lut.shape
    assert B % NW == 0 and S % GCHUNK == 0 and D == D_MODEL
    rows_per_w = B // NW

    lut_view = jnp.pad(lut, ((0, 0), (0, D_PAD - D))).reshape(2 * V, D)
    x2 = x * 2

    scratch = [pltpu.VMEM((rows_per_w, S), jnp.int32)]
    scratch += [pltpu.VMEM((S, D), jnp.float32) for _ in range(NBUF)]
    scratch += [pltpu.SemaphoreType.DMA for _ in range(1 + 2 * NBUF)]

    mesh = plsc.VectorSubcoreMesh(core_axis_name="c", subcore_axis_name="s")
    k = functools.partial(
        pl.kernel,
        mesh=mesh,
        out_type=jax.ShapeDtypeStruct((B, S, D_PAD), jnp.float32),
        scratch_types=scratch,
        compiler_params=pltpu.CompilerParams(use_tc_tiling_on_sc=False),
    )(functools.partial(_emb_body, rows_per_w, S))

    out = k(lut_view, x2)
    return out[..., :D_MODEL]
